# Initial kernel scaffold; baseline (speedup 1.0000x reference)
import jax
import jax.numpy as jnp
from jax.experimental import pallas as pl


def kernel(adj_indices, adj_values, user_emb, item_emb):
    raise NotImplementedError("write your pallas kernel here")


# speedup vs baseline: 3.4576x; 3.4576x over previous
import functools

import jax
import jax.numpy as jnp
from jax import lax
from jax.experimental import pallas as pl
from jax.experimental.pallas import tpu as pltpu
from jax.experimental.pallas import tpu_sc as plsc

N_USER = 10000
N_ITEM = 40000
N_NODES = N_USER + N_ITEM
D = 64
HALF = 32
N_EDGES = 800000
NC = 2
NS = 16
E_TILE = N_EDGES // NS
CHUNK = 128
N_CHUNK = E_TILE // CHUNK
TAIL = E_TILE - N_CHUNK * CHUNK
RCHUNK = 200
N_RCHUNK = N_NODES // RCHUNK


def _body(src, dst, vals, ego, zeros_hbm, final_out, e1_out, e2_out,
          acc, src_v, gidx_v, dst_v, val_v, rows_v,
          gidx_t, dst_t, val_t, b1, b2, b3, sem):
  c = lax.axis_index("c")
  s = lax.axis_index("s")
  coff = c * N_NODES
  tbase = s * E_TILE
  third = jnp.float32(1.0 / 3.0)

  def scale_rows(val_ref, n_edges):
    def scale(g, _):
      v16 = val_ref[pl.ds(g * 16, 16)]
      for l in range(16):
        e = g * 16 + l
        v = v16[l]
        rows_v[e, pl.ds(0, 16)] = rows_v[e, pl.ds(0, 16)] * v
        rows_v[e, pl.ds(16, 16)] = rows_v[e, pl.ds(16, 16)] * v
      return 0
    lax.fori_loop(0, n_edges // 16, scale, 0)

  n_rc = 15 + jnp.where(s < N_RCHUNK - 15 * NS, 1, 0)

  def layer(tab_in, tab_out, last):
    def zero_body(k, _):
      r0 = (s + k * NS) * RCHUNK
      pltpu.sync_copy(zeros_hbm, acc.at[pl.ds(r0, RCHUNK)])
      return 0
    lax.fori_loop(0, n_rc, zero_body, 0)
    plsc.subcore_barrier()

    def chunk_body(i, _):
      b = tbase + i * CHUNK
      pltpu.sync_copy(src.at[pl.ds(b, CHUNK)], src_v)
      pltpu.sync_copy(dst.at[pl.ds(b, CHUNK)], dst_v)
      pltpu.sync_copy(vals.at[pl.ds(b, CHUNK)], val_v)
      for j in range(CHUNK // 16):
        gidx_v[pl.ds(j * 16, 16)] = src_v[pl.ds(j * 16, 16)] + coff
      pltpu.async_copy(tab_in.at[gidx_v], rows_v, sem).wait()
      scale_rows(val_v, CHUNK)
      pltpu.sync_copy(rows_v, acc.at[dst_v], add=True)
      return 0
    lax.fori_loop(0, N_CHUNK, chunk_body, 0)

    b = tbase + N_CHUNK * CHUNK
    pltpu.sync_copy(src.at[pl.ds(b, TAIL)], dst_t)
    pltpu.sync_copy(vals.at[pl.ds(b, TAIL)], val_t)
    for j in range(TAIL // 16):
      gidx_t[pl.ds(j * 16, 16)] = dst_t[pl.ds(j * 16, 16)] + coff
    pltpu.sync_copy(dst.at[pl.ds(b, TAIL)], dst_t)
    pltpu.async_copy(tab_in.at[gidx_t], rows_v.at[pl.ds(0, TAIL)], sem).wait()
    scale_rows(val_t, TAIL)
    pltpu.sync_copy(rows_v.at[pl.ds(0, TAIL)], acc.at[dst_t], add=True)

    plsc.subcore_barrier()
    if not last:
      def pub_body(k, _):
        r0 = (s + k * NS) * RCHUNK
        pltpu.sync_copy(acc.at[pl.ds(r0, RCHUNK)],
                        tab_out.at[pl.ds(coff + r0, RCHUNK)])
        return 0
      lax.fori_loop(0, n_rc, pub_body, 0)
      plsc.subcore_barrier()
    else:
      def mean_body(k, _):
        r0 = (s + k * NS) * RCHUNK
        pltpu.sync_copy(e1_out.at[pl.ds(coff + r0, RCHUNK)], b1)
        pltpu.sync_copy(e2_out.at[pl.ds(coff + r0, RCHUNK)], b2)
        pltpu.sync_copy(acc.at[pl.ds(r0, RCHUNK)], b3)
        def mrow(rr, _):
          for j in range(HALF // 16):
            sl = pl.ds(j * 16, 16)
            b1[rr, sl] = (b1[rr, sl] + b2[rr, sl] + b3[rr, sl]) * third
          return 0
        lax.fori_loop(0, RCHUNK, mrow, 0)
        pltpu.sync_copy(b1, final_out.at[pl.ds(coff + r0, RCHUNK)])
        return 0
      lax.fori_loop(0, n_rc, mean_body, 0)

  layer(ego, e1_out, False)
  layer(e1_out, e2_out, False)
  layer(e2_out, None, True)


@jax.jit
def _run(src, dst, adj_values, ego_half, zeros_hbm):
  mesh = plsc.VectorSubcoreMesh(core_axis_name="c", subcore_axis_name="s")
  f = pl.kernel(
      _body,
      out_type=[
          jax.ShapeDtypeStruct((NC * N_NODES, HALF), jnp.float32),
          jax.ShapeDtypeStruct((NC * N_NODES, HALF), jnp.float32),
          jax.ShapeDtypeStruct((NC * N_NODES, HALF), jnp.float32),
      ],
      mesh=mesh,
      compiler_params=pltpu.CompilerParams(use_tc_tiling_on_sc=False),
      scratch_types=[
          pltpu.VMEM_SHARED((N_NODES, HALF), jnp.float32),
          pltpu.VMEM((CHUNK,), jnp.int32),
          pltpu.VMEM((CHUNK,), jnp.int32),
          pltpu.VMEM((CHUNK,), jnp.int32),
          pltpu.VMEM((CHUNK,), jnp.float32),
          pltpu.VMEM((CHUNK, HALF), jnp.float32),
          pltpu.VMEM((TAIL,), jnp.int32),
          pltpu.VMEM((TAIL,), jnp.int32),
          pltpu.VMEM((TAIL,), jnp.float32),
          pltpu.VMEM((RCHUNK, HALF), jnp.float32),
          pltpu.VMEM((RCHUNK, HALF), jnp.float32),
          pltpu.VMEM((RCHUNK, HALF), jnp.float32),
          pltpu.SemaphoreType.DMA,
      ],
  )
  final, _, _ = f(src, dst, adj_values, ego_half, zeros_hbm)
  return final


def kernel(adj_indices, adj_values, user_emb, item_emb):
  ego = jnp.concatenate([user_emb, item_emb], axis=0)
  ego_half = jnp.concatenate([ego[:, :HALF], ego[:, HALF:]], axis=0)
  zeros_hbm = jnp.zeros((RCHUNK, HALF), jnp.float32)
  half = _run(adj_indices[0], adj_indices[1], adj_values, ego_half, zeros_hbm)
  final = jnp.concatenate([half[:N_NODES], half[N_NODES:]], axis=1)
  return (final[:N_USER], final[N_USER:])


# speedup vs baseline: 6.5101x; 1.8829x over previous
import jax
import jax.numpy as jnp
from jax import lax
from jax.experimental import pallas as pl
from jax.experimental.pallas import tpu as pltpu
from jax.experimental.pallas import tpu_sc as plsc

N_USER = 10000
N_ITEM = 40000
N_NODES = N_USER + N_ITEM
D = 64
HALF = 32
N_EDGES = 800000
NC = 2
NS = 16
E_TILE = N_EDGES // NS
SB_E = 1024
E_PAD = 50176
N_SB = E_PAD // SB_E
CHUNK = 128
SB_CH = SB_E // CHUNK
N_CHUNK = E_PAD // CHUNK
NBUF = 4
RCHUNK = 80
N_RCHUNK = N_NODES // RCHUNK


def _body(src, dst, vals, ego, zeros_hbm, final_out, e1_out, e2_out,
          acc, srcB, dstB, valB, gidx, didx, rows,
          sem_m, sem_g, sem_s):
  c = lax.axis_index("c")
  s = lax.axis_index("s")
  coff = c * N_NODES
  tbase = s * E_PAD
  third = jnp.float32(1.0 / 3.0)

  n_rc = (N_RCHUNK // NS) + jnp.where(s < N_RCHUNK % NS, 1, 0)

  def meta_args(sb, parity):
    e0 = tbase + sb * SB_E
    o = parity * SB_E
    return (
        (src.at[pl.ds(e0, SB_E)], srcB.at[pl.ds(o, SB_E)], sem_m.at[parity]),
        (dst.at[pl.ds(e0, SB_E)], dstB.at[pl.ds(o, SB_E)], sem_m.at[parity]),
        (vals.at[pl.ds(e0, SB_E)], valB.at[pl.ds(o, SB_E)], sem_m.at[parity]),
    )

  def issue_meta(sb, parity):
    for a in meta_args(sb, parity):
      pltpu.async_copy(*a)

  def wait_meta(sb, parity):
    for a in meta_args(sb, parity):
      pltpu.make_async_copy(*a).wait()

  def layer(tab_in, tab_out, last):
    def zero_body(k, _):
      r0 = (s + k * NS) * RCHUNK
      pltpu.sync_copy(zeros_hbm, acc.at[pl.ds(r0, RCHUNK)])
      return 0
    lax.fori_loop(0, n_rc, zero_body, 0)
    plsc.subcore_barrier()

    def start_gather(b):
      pltpu.async_copy(tab_in.at[gidx.at[b]], rows.at[b], sem_g.at[b])

    def wait_gather(b):
      pltpu.make_async_copy(tab_in.at[gidx.at[b]], rows.at[b],
                            sem_g.at[b]).wait()

    def start_scatter(b):
      pltpu.async_copy(rows.at[b], acc.at[didx.at[b]], sem_s.at[b], add=True)

    def wait_scatter(b):
      pltpu.make_async_copy(rows.at[b], acc.at[didx.at[b]],
                            sem_s.at[b]).wait()

    def stage_a(i):
      b = i % NBUF
      sb = i // SB_CH
      moff = (sb % 2) * SB_E + (i % SB_CH) * CHUNK
      for j in range(CHUNK // 16):
        sl = pl.ds(j * 16, 16)
        gidx[b, sl] = srcB[pl.ds(moff + j * 16, 16)] + coff
        didx[b, sl] = dstB[pl.ds(moff + j * 16, 16)]
      start_gather(b)

    def stage_b(i):
      b = i % NBUF
      sb = i // SB_CH
      moff = (sb % 2) * SB_E + (i % SB_CH) * CHUNK
      wait_gather(b)
      def scale(g, _):
        v16 = valB[pl.ds(moff + g * 16, 16)]
        for l in range(16):
          e = g * 16 + l
          v = v16[l]
          rows[b, e, pl.ds(0, 16)] = rows[b, e, pl.ds(0, 16)] * v
          rows[b, e, pl.ds(16, 16)] = rows[b, e, pl.ds(16, 16)] * v
        return 0
      lax.fori_loop(0, CHUNK // 16, scale, 0)
      start_scatter(b)

    issue_meta(0, 0)

    def chunk_body(i, _):
      sb = i // SB_CH
      parity = sb % 2
      @pl.when(i >= 1)
      def _():
        stage_b(i - 1)
      @pl.when(i % SB_CH == 0)
      def _():
        @pl.when(sb + 1 < N_SB)
        def _():
          issue_meta(sb + 1, 1 - parity)
        wait_meta(sb, parity)
      @pl.when(i >= NBUF)
      def _():
        wait_scatter(i % NBUF)
      stage_a(i)
      return 0
    lax.fori_loop(0, N_CHUNK, chunk_body, 0)
    stage_b(N_CHUNK - 1)
    for b in range(NBUF):
      wait_scatter(b)

    plsc.subcore_barrier()
    if not last:
      def pub_body(k, _):
        r0 = (s + k * NS) * RCHUNK
        pltpu.sync_copy(acc.at[pl.ds(r0, RCHUNK)],
                        tab_out.at[pl.ds(coff + r0, RCHUNK)])
        return 0
      lax.fori_loop(0, n_rc, pub_body, 0)
      plsc.subcore_barrier()
    else:
      def mean_body(k, _):
        r0 = (s + k * NS) * RCHUNK
        pltpu.sync_copy(e1_out.at[pl.ds(coff + r0, RCHUNK)],
                        rows.at[0, pl.ds(0, RCHUNK)])
        pltpu.sync_copy(e2_out.at[pl.ds(coff + r0, RCHUNK)],
                        rows.at[1, pl.ds(0, RCHUNK)])
        pltpu.sync_copy(acc.at[pl.ds(r0, RCHUNK)],
                        rows.at[2, pl.ds(0, RCHUNK)])
        def mrow(rr, _):
          for j in range(HALF // 16):
            sl = pl.ds(j * 16, 16)
            rows[0, rr, sl] = (rows[0, rr, sl] + rows[1, rr, sl]
                               + rows[2, rr, sl]) * third
          return 0
        lax.fori_loop(0, RCHUNK, mrow, 0)
        pltpu.sync_copy(rows.at[0, pl.ds(0, RCHUNK)],
                        final_out.at[pl.ds(coff + r0, RCHUNK)])
        return 0
      lax.fori_loop(0, n_rc, mean_body, 0)

  layer(ego, e1_out, False)
  layer(e1_out, e2_out, False)
  layer(e2_out, None, True)


@jax.jit
def _run(src, dst, adj_values, ego_half, zeros_hbm):
  mesh = plsc.VectorSubcoreMesh(core_axis_name="c", subcore_axis_name="s")
  f = pl.kernel(
      _body,
      out_type=[
          jax.ShapeDtypeStruct((NC * N_NODES, HALF), jnp.float32),
          jax.ShapeDtypeStruct((NC * N_NODES, HALF), jnp.float32),
          jax.ShapeDtypeStruct((NC * N_NODES, HALF), jnp.float32),
      ],
      mesh=mesh,
      compiler_params=pltpu.CompilerParams(use_tc_tiling_on_sc=False),
      scratch_types=[
          pltpu.VMEM_SHARED((N_NODES, HALF), jnp.float32),
          pltpu.VMEM((2 * SB_E,), jnp.int32),
          pltpu.VMEM((2 * SB_E,), jnp.int32),
          pltpu.VMEM((2 * SB_E,), jnp.float32),
          pltpu.VMEM((NBUF, CHUNK), jnp.int32),
          pltpu.VMEM((NBUF, CHUNK), jnp.int32),
          pltpu.VMEM((NBUF, CHUNK, HALF), jnp.float32),
          pltpu.SemaphoreType.DMA((2,)),
          pltpu.SemaphoreType.DMA((NBUF,)),
          pltpu.SemaphoreType.DMA((NBUF,)),
      ],
  )
  final, _, _ = f(src, dst, adj_values, ego_half, zeros_hbm)
  return final


def _pad_edges(x):
  return jnp.pad(x.reshape(NS, E_TILE),
                 ((0, 0), (0, E_PAD - E_TILE))).reshape(-1)


def kernel(adj_indices, adj_values, user_emb, item_emb):
  ego = jnp.concatenate([user_emb, item_emb], axis=0)
  ego_half = jnp.concatenate([ego[:, :HALF], ego[:, HALF:]], axis=0)
  zeros_hbm = jnp.zeros((RCHUNK, HALF), jnp.float32)
  srcp = _pad_edges(adj_indices[0])
  dstp = _pad_edges(adj_indices[1])
  valp = _pad_edges(adj_values)
  half = _run(srcp, dstp, valp, ego_half, zeros_hbm)
  final = jnp.concatenate([half[:N_NODES], half[N_NODES:]], axis=1)
  return (final[:N_USER], final[N_USER:])


# speedup vs baseline: 13.4861x; 2.0716x over previous
import jax
import jax.numpy as jnp
from jax import lax
from jax.experimental import pallas as pl
from jax.experimental.pallas import tpu as pltpu
from jax.experimental.pallas import tpu_sc as plsc

N_USER = 10000
N_ITEM = 40000
N_NODES = N_USER + N_ITEM
D = 64
HALF = 32
N_EDGES = 800000
NC = 2
NS = 16
E_TILE = N_EDGES // NS
SB_E = 512
E_PAD = 50176
N_SB = E_PAD // SB_E
CHUNK = 128
SB_CH = SB_E // CHUNK
N_CHUNK = E_PAD // CHUNK
NBUF = 5
LA = 3
NMETA = 3
RCHUNK = 80
N_RCHUNK = N_NODES // RCHUNK
RC_Z = 400
N_RC_Z = N_NODES // RC_Z


def _body(src, dst, vals, ego, zeros_hbm, final_out, e1_out, e2_out,
          acc, srcB, dstB, valB, gidx, didx, rows,
          sem_m, sem_g, sem_s):
  c = lax.axis_index("c")
  s = lax.axis_index("s")
  coff = c * N_NODES
  tbase = s * E_PAD
  third = jnp.float32(1.0 / 3.0)

  n_rc = (N_RCHUNK // NS) + jnp.where(s < N_RCHUNK % NS, 1, 0)
  n_rcz = (N_RC_Z // NS) + jnp.where(s < N_RC_Z % NS, 1, 0)

  def meta_args(sb, parity):
    e0 = tbase + sb * SB_E
    o = parity * SB_E
    return (
        (src.at[pl.ds(e0, SB_E)], srcB.at[pl.ds(o, SB_E)], sem_m.at[parity]),
        (dst.at[pl.ds(e0, SB_E)], dstB.at[pl.ds(o, SB_E)], sem_m.at[parity]),
        (vals.at[pl.ds(e0, SB_E)], valB.at[pl.ds(o, SB_E)], sem_m.at[parity]),
    )

  def issue_meta(sb, parity):
    for a in meta_args(sb, parity):
      pltpu.async_copy(*a)

  def wait_meta(sb, parity):
    for a in meta_args(sb, parity):
      pltpu.make_async_copy(*a).wait()

  def layer(tab_in, tab_out, last):
    def zero_body(k, _):
      r0 = (s + k * NS) * RC_Z
      pltpu.sync_copy(zeros_hbm, acc.at[pl.ds(r0, RC_Z)])
      return 0
    lax.fori_loop(0, n_rcz, zero_body, 0)
    plsc.subcore_barrier()

    def start_gather(b):
      pltpu.async_copy(tab_in.at[gidx.at[b]], rows.at[b], sem_g.at[b])

    def wait_gather(b):
      pltpu.make_async_copy(tab_in.at[gidx.at[b]], rows.at[b],
                            sem_g.at[b]).wait()

    def start_scatter(b):
      pltpu.async_copy(rows.at[b], acc.at[didx.at[b]], sem_s.at[b], add=True)

    def wait_scatter(b):
      pltpu.make_async_copy(rows.at[b], acc.at[didx.at[b]],
                            sem_s.at[b]).wait()

    def stage_a(i):
      b = i % NBUF
      sb = i // SB_CH
      moff = (sb % NMETA) * SB_E + (i % SB_CH) * CHUNK
      for j in range(CHUNK // 16):
        sl = pl.ds(j * 16, 16)
        gidx[b, sl] = srcB[pl.ds(moff + j * 16, 16)] + coff
        didx[b, sl] = dstB[pl.ds(moff + j * 16, 16)]
      start_gather(b)

    def stage_b(i):
      b = i % NBUF
      sb = i // SB_CH
      moff = (sb % NMETA) * SB_E + (i % SB_CH) * CHUNK
      wait_gather(b)
      def scale(g, _):
        v16 = valB[pl.ds(moff + g * 16, 16)]
        for l in range(16):
          e = g * 16 + l
          v = v16[l]
          rows[b, e, pl.ds(0, 16)] = rows[b, e, pl.ds(0, 16)] * v
          rows[b, e, pl.ds(16, 16)] = rows[b, e, pl.ds(16, 16)] * v
        return 0
      lax.fori_loop(0, CHUNK // 16, scale, 0)
      start_scatter(b)

    issue_meta(0, 0)

    def chunk_body(i, _):
      sb = i // SB_CH
      @pl.when(i % SB_CH == 0)
      def _():
        @pl.when(sb + 1 < N_SB)
        def _():
          issue_meta(sb + 1, (sb + 1) % NMETA)
        wait_meta(sb, sb % NMETA)
      @pl.when(i >= NBUF)
      def _():
        wait_scatter(i % NBUF)
      stage_a(i)
      @pl.when(i >= LA)
      def _():
        stage_b(i - LA)
      return 0
    lax.fori_loop(0, N_CHUNK, chunk_body, 0)
    for k in range(LA):
      stage_b(N_CHUNK - LA + k)
    for b in range(NBUF):
      wait_scatter(b)

    plsc.subcore_barrier()
    if not last:
      def pub_body(k, _):
        r0 = (s + k * NS) * RC_Z
        pltpu.sync_copy(acc.at[pl.ds(r0, RC_Z)],
                        tab_out.at[pl.ds(coff + r0, RC_Z)])
        return 0
      lax.fori_loop(0, n_rcz, pub_body, 0)
      plsc.subcore_barrier()
    else:
      def mean_body(k, _):
        r0 = (s + k * NS) * RCHUNK
        pltpu.sync_copy(e1_out.at[pl.ds(coff + r0, RCHUNK)],
                        rows.at[0, pl.ds(0, RCHUNK)])
        pltpu.sync_copy(e2_out.at[pl.ds(coff + r0, RCHUNK)],
                        rows.at[1, pl.ds(0, RCHUNK)])
        pltpu.sync_copy(acc.at[pl.ds(r0, RCHUNK)],
                        rows.at[2, pl.ds(0, RCHUNK)])
        def mrow(rr, _):
          for j in range(HALF // 16):
            sl = pl.ds(j * 16, 16)
            rows[0, rr, sl] = (rows[0, rr, sl] + rows[1, rr, sl]
                               + rows[2, rr, sl]) * third
          return 0
        lax.fori_loop(0, RCHUNK, mrow, 0)
        pltpu.sync_copy(rows.at[0, pl.ds(0, RCHUNK)],
                        final_out.at[pl.ds(coff + r0, RCHUNK)])
        return 0
      lax.fori_loop(0, n_rc, mean_body, 0)

  layer(ego, e1_out, False)
  layer(e1_out, e2_out, False)
  layer(e2_out, None, True)


@jax.jit
def _run(src, dst, adj_values, ego_half, zeros_hbm):
  mesh = plsc.VectorSubcoreMesh(core_axis_name="c", subcore_axis_name="s")
  f = pl.kernel(
      _body,
      out_type=[
          jax.ShapeDtypeStruct((NC * N_NODES, HALF), jnp.float32),
          jax.ShapeDtypeStruct((NC * N_NODES, HALF), jnp.float32),
          jax.ShapeDtypeStruct((NC * N_NODES, HALF), jnp.float32),
      ],
      mesh=mesh,
      compiler_params=pltpu.CompilerParams(use_tc_tiling_on_sc=False),
      scratch_types=[
          pltpu.VMEM_SHARED((N_NODES, HALF), jnp.float32),
          pltpu.VMEM((NMETA * SB_E,), jnp.int32),
          pltpu.VMEM((NMETA * SB_E,), jnp.int32),
          pltpu.VMEM((NMETA * SB_E,), jnp.float32),
          pltpu.VMEM((NBUF, CHUNK), jnp.int32),
          pltpu.VMEM((NBUF, CHUNK), jnp.int32),
          pltpu.VMEM((NBUF, CHUNK, HALF), jnp.float32),
          pltpu.SemaphoreType.DMA((NMETA,)),
          pltpu.SemaphoreType.DMA((NBUF,)),
          pltpu.SemaphoreType.DMA((NBUF,)),
      ],
  )
  final, _, _ = f(src, dst, adj_values, ego_half, zeros_hbm)
  return final


def _pad_edges(x):
  return jnp.pad(x.reshape(NS, E_TILE),
                 ((0, 0), (0, E_PAD - E_TILE))).reshape(-1)


def kernel(adj_indices, adj_values, user_emb, item_emb):
  ego = jnp.concatenate([user_emb, item_emb], axis=0)
  ego_half = jnp.concatenate([ego[:, :HALF], ego[:, HALF:]], axis=0)
  zeros_hbm = jnp.zeros((RC_Z, HALF), jnp.float32)
  srcp = _pad_edges(adj_indices[0])
  dstp = _pad_edges(adj_indices[1])
  valp = _pad_edges(adj_values)
  half = _run(srcp, dstp, valp, ego_half, zeros_hbm)
  final = jnp.concatenate([half[:N_NODES], half[N_NODES:]], axis=1)
  return (final[:N_USER], final[N_USER:])


# speedup vs baseline: 13.5961x; 1.0082x over previous
import jax
import jax.numpy as jnp
from jax import lax
from jax.experimental import pallas as pl
from jax.experimental.pallas import tpu as pltpu
from jax.experimental.pallas import tpu_sc as plsc

N_USER = 10000
N_ITEM = 40000
N_NODES = N_USER + N_ITEM
D = 64
HALF = 32
N_EDGES = 800000
NC = 2
NS = 16
E_TILE = N_EDGES // NS
SB_E = 512
E_PAD = 50176
N_SB = E_PAD // SB_E
CHUNK = 128
SB_CH = SB_E // CHUNK
N_CHUNK = E_PAD // CHUNK
NBUF = 6
LA = 4
NMETA = 3
RCHUNK = 80
N_RCHUNK = N_NODES // RCHUNK
RC_Z = 400
N_RC_Z = N_NODES // RC_Z


def _body(src, dst, vals, ego, zeros_hbm, final_out, e1_out, e2_out,
          acc, srcB, dstB, valB, gidx, didx, rows,
          sem_m, sem_g, sem_s):
  c = lax.axis_index("c")
  s = lax.axis_index("s")
  coff = c * N_NODES
  tbase = s * E_PAD
  third = jnp.float32(1.0 / 3.0)

  n_rc = (N_RCHUNK // NS) + jnp.where(s < N_RCHUNK % NS, 1, 0)
  n_rcz = (N_RC_Z // NS) + jnp.where(s < N_RC_Z % NS, 1, 0)

  def meta_args(sb, parity):
    e0 = tbase + sb * SB_E
    o = parity * SB_E
    return (
        (src.at[pl.ds(e0, SB_E)], srcB.at[pl.ds(o, SB_E)], sem_m.at[parity]),
        (dst.at[pl.ds(e0, SB_E)], dstB.at[pl.ds(o, SB_E)], sem_m.at[parity]),
        (vals.at[pl.ds(e0, SB_E)], valB.at[pl.ds(o, SB_E)], sem_m.at[parity]),
    )

  def issue_meta(sb, parity):
    for a in meta_args(sb, parity):
      pltpu.async_copy(*a)

  def wait_meta(sb, parity):
    for a in meta_args(sb, parity):
      pltpu.make_async_copy(*a).wait()

  def layer(tab_in, tab_out, last):
    def zero_body(k, _):
      r0 = (s + k * NS) * RC_Z
      pltpu.sync_copy(zeros_hbm, acc.at[pl.ds(r0, RC_Z)])
      return 0
    lax.fori_loop(0, n_rcz, zero_body, 0)
    plsc.subcore_barrier()

    def start_gather(b):
      pltpu.async_copy(tab_in.at[gidx.at[b]], rows.at[b], sem_g.at[b])

    def wait_gather(b):
      pltpu.make_async_copy(tab_in.at[gidx.at[b]], rows.at[b],
                            sem_g.at[b]).wait()

    def start_scatter(b):
      pltpu.async_copy(rows.at[b], acc.at[didx.at[b]], sem_s.at[b], add=True)

    def wait_scatter(b):
      pltpu.make_async_copy(rows.at[b], acc.at[didx.at[b]],
                            sem_s.at[b]).wait()

    def stage_a(i):
      b = i % NBUF
      sb = i // SB_CH
      moff = (sb % NMETA) * SB_E + (i % SB_CH) * CHUNK
      for j in range(CHUNK // 16):
        sl = pl.ds(j * 16, 16)
        gidx[b, sl] = srcB[pl.ds(moff + j * 16, 16)] + coff
        didx[b, sl] = dstB[pl.ds(moff + j * 16, 16)]
      start_gather(b)

    def stage_b(i):
      b = i % NBUF
      sb = i // SB_CH
      moff = (sb % NMETA) * SB_E + (i % SB_CH) * CHUNK
      wait_gather(b)
      def scale(g, _):
        v16 = valB[pl.ds(moff + g * 16, 16)]
        for l in range(16):
          e = g * 16 + l
          v = v16[l]
          rows[b, e, pl.ds(0, 16)] = rows[b, e, pl.ds(0, 16)] * v
          rows[b, e, pl.ds(16, 16)] = rows[b, e, pl.ds(16, 16)] * v
        return 0
      lax.fori_loop(0, CHUNK // 16, scale, 0)
      start_scatter(b)

    issue_meta(0, 0)

    def chunk_body(i, _):
      sb = i // SB_CH
      @pl.when(i % SB_CH == 0)
      def _():
        @pl.when(sb + 1 < N_SB)
        def _():
          issue_meta(sb + 1, (sb + 1) % NMETA)
        wait_meta(sb, sb % NMETA)
      @pl.when(i >= NBUF)
      def _():
        wait_scatter(i % NBUF)
      stage_a(i)
      @pl.when(i >= LA)
      def _():
        stage_b(i - LA)
      return 0
    lax.fori_loop(0, N_CHUNK, chunk_body, 0)
    for k in range(LA):
      stage_b(N_CHUNK - LA + k)
    for b in range(NBUF):
      wait_scatter(b)

    plsc.subcore_barrier()
    if not last:
      def pub_body(k, _):
        r0 = (s + k * NS) * RC_Z
        pltpu.sync_copy(acc.at[pl.ds(r0, RC_Z)],
                        tab_out.at[pl.ds(coff + r0, RC_Z)])
        return 0
      lax.fori_loop(0, n_rcz, pub_body, 0)
      plsc.subcore_barrier()
    else:
      def mean_body(k, _):
        r0 = (s + k * NS) * RCHUNK
        pltpu.sync_copy(e1_out.at[pl.ds(coff + r0, RCHUNK)],
                        rows.at[0, pl.ds(0, RCHUNK)])
        pltpu.sync_copy(e2_out.at[pl.ds(coff + r0, RCHUNK)],
                        rows.at[1, pl.ds(0, RCHUNK)])
        pltpu.sync_copy(acc.at[pl.ds(r0, RCHUNK)],
                        rows.at[2, pl.ds(0, RCHUNK)])
        def mrow(rr, _):
          for j in range(HALF // 16):
            sl = pl.ds(j * 16, 16)
            rows[0, rr, sl] = (rows[0, rr, sl] + rows[1, rr, sl]
                               + rows[2, rr, sl]) * third
          return 0
        lax.fori_loop(0, RCHUNK, mrow, 0)
        pltpu.sync_copy(rows.at[0, pl.ds(0, RCHUNK)],
                        final_out.at[pl.ds(coff + r0, RCHUNK)])
        return 0
      lax.fori_loop(0, n_rc, mean_body, 0)

  layer(ego, e1_out, False)
  layer(e1_out, e2_out, False)
  layer(e2_out, None, True)


@jax.jit
def _run(src, dst, adj_values, ego_half, zeros_hbm):
  mesh = plsc.VectorSubcoreMesh(core_axis_name="c", subcore_axis_name="s")
  f = pl.kernel(
      _body,
      out_type=[
          jax.ShapeDtypeStruct((NC * N_NODES, HALF), jnp.float32),
          jax.ShapeDtypeStruct((NC * N_NODES, HALF), jnp.float32),
          jax.ShapeDtypeStruct((NC * N_NODES, HALF), jnp.float32),
      ],
      mesh=mesh,
      compiler_params=pltpu.CompilerParams(use_tc_tiling_on_sc=False),
      scratch_types=[
          pltpu.VMEM_SHARED((N_NODES, HALF), jnp.float32),
          pltpu.VMEM((NMETA * SB_E,), jnp.int32),
          pltpu.VMEM((NMETA * SB_E,), jnp.int32),
          pltpu.VMEM((NMETA * SB_E,), jnp.float32),
          pltpu.VMEM((NBUF, CHUNK), jnp.int32),
          pltpu.VMEM((NBUF, CHUNK), jnp.int32),
          pltpu.VMEM((NBUF, CHUNK, HALF), jnp.float32),
          pltpu.SemaphoreType.DMA((NMETA,)),
          pltpu.SemaphoreType.DMA((NBUF,)),
          pltpu.SemaphoreType.DMA((NBUF,)),
      ],
  )
  final, _, _ = f(src, dst, adj_values, ego_half, zeros_hbm)
  return final


def _pad_edges(x):
  return jnp.pad(x.reshape(NS, E_TILE),
                 ((0, 0), (0, E_PAD - E_TILE))).reshape(-1)


def kernel(adj_indices, adj_values, user_emb, item_emb):
  ego = jnp.concatenate([user_emb, item_emb], axis=0)
  ego_half = jnp.concatenate([ego[:, :HALF], ego[:, HALF:]], axis=0)
  zeros_hbm = jnp.zeros((RC_Z, HALF), jnp.float32)
  srcp = _pad_edges(adj_indices[0])
  dstp = _pad_edges(adj_indices[1])
  valp = _pad_edges(adj_values)
  half = _run(srcp, dstp, valp, ego_half, zeros_hbm)
  final = jnp.concatenate([half[:N_NODES], half[N_NODES:]], axis=1)
  return (final[:N_USER], final[N_USER:])
